# Initial kernel scaffold; baseline (speedup 1.0000x reference)
import functools

import jax
import jax.numpy as jnp
from jax import lax
from jax.experimental import pallas as pl
from jax.experimental.pallas import tpu as pltpu
from jax.experimental.pallas import tpu_sc as plsc

N = 10000
E = 320000
D = 128

NC = 2
NS = 16
NW = NC * NS

EPT = E // NW
CH = 80
NCHUNK = EPT // CH
RPT = N // NS
ZR = 125

_mesh = plsc.VectorSubcoreMesh(core_axis_name="c", subcore_axis_name="s")


def _sc_degrees(src, dst):
    @functools.partial(
        pl.kernel,
        out_type=[
            jax.ShapeDtypeStruct((NC, N, 16), jnp.float32),
            jax.ShapeDtypeStruct((NC, N, 16), jnp.float32),
        ],
        mesh=_mesh,
        scratch_types=[
            pltpu.VMEM((CH,), jnp.int32),
            pltpu.VMEM((CH,), jnp.int32),
            pltpu.VMEM((CH, 16), jnp.float32),
            pltpu.VMEM((RPT, 16), jnp.float32),
            pltpu.VMEM_SHARED((N, 16), jnp.float32),
            pltpu.VMEM_SHARED((N, 16), jnp.float32),
        ],
    )
    def k(src_h, dst_h, dop_h, dip_h, sidx, didx, ones_v, zeros_v, acc_o, acc_i):
        c = lax.axis_index("c")
        s = lax.axis_index("s")
        wid = c * NS + s
        ebase = wid * EPT
        r0 = s * RPT

        one16 = jnp.full((16,), 1.0, jnp.float32)
        zero16 = jnp.zeros((16,), jnp.float32)

        @pl.loop(0, CH)
        def _(i):
            ones_v[i, :] = one16

        @pl.loop(0, RPT)
        def _(i):
            zeros_v[i, :] = zero16

        pltpu.sync_copy(zeros_v, acc_o.at[pl.ds(r0, RPT)])
        pltpu.sync_copy(zeros_v, acc_i.at[pl.ds(r0, RPT)])
        plsc.subcore_barrier()

        @pl.loop(0, NCHUNK)
        def _(ci):
            base = ebase + ci * CH
            pltpu.sync_copy(src_h.at[pl.ds(base, CH)], sidx)
            pltpu.sync_copy(dst_h.at[pl.ds(base, CH)], didx)
            pltpu.sync_copy(ones_v, acc_o.at[sidx], add=True)
            pltpu.sync_copy(ones_v, acc_i.at[didx], add=True)

        plsc.subcore_barrier()
        pltpu.sync_copy(acc_o.at[pl.ds(r0, RPT)], dop_h.at[c, pl.ds(r0, RPT), :])
        pltpu.sync_copy(acc_i.at[pl.ds(r0, RPT)], dip_h.at[c, pl.ds(r0, RPT), :])

    return k(src, dst)


def _sc_agg(h, src, dst):
    @functools.partial(
        pl.kernel,
        out_type=jax.ShapeDtypeStruct((NC, N, D), jnp.float32),
        mesh=_mesh,
        scratch_types=[
            pltpu.VMEM((CH,), jnp.int32),
            pltpu.VMEM((CH,), jnp.int32),
            pltpu.VMEM((CH, D), jnp.float32),
            pltpu.VMEM((ZR, D), jnp.float32),
            pltpu.VMEM_SHARED((N, D), jnp.float32),
            pltpu.SemaphoreType.DMA,
        ],
    )
    def k(h_h, src_h, dst_h, out_h, sidx, didx, rows, zeros_v, acc, sem):
        c = lax.axis_index("c")
        s = lax.axis_index("s")
        wid = c * NS + s
        ebase = wid * EPT
        r0 = s * RPT

        zero16 = jnp.zeros((16,), jnp.float32)

        @pl.loop(0, ZR)
        def _(i):
            @pl.loop(0, D, step=16)
            def _(j):
                zeros_v[i, pl.ds(j, 16)] = zero16

        @pl.loop(0, RPT, step=ZR)
        def _(r):
            pltpu.sync_copy(zeros_v, acc.at[pl.ds(r0 + r, ZR)])

        plsc.subcore_barrier()

        @pl.loop(0, NCHUNK)
        def _(ci):
            base = ebase + ci * CH
            pltpu.sync_copy(src_h.at[pl.ds(base, CH)], sidx)
            pltpu.sync_copy(dst_h.at[pl.ds(base, CH)], didx)
            pltpu.async_copy(h_h.at[sidx], rows, sem).wait()
            pltpu.sync_copy(rows, acc.at[didx], add=True)

        plsc.subcore_barrier()
        pltpu.sync_copy(acc.at[pl.ds(r0, RPT)], out_h.at[c, pl.ds(r0, RPT), :])

    return k(h, src, dst)


BLK = 1024
GRID = (N + BLK - 1) // BLK


def _norm_from_partials(p_ref):
    d = p_ref[0] + p_ref[1]
    deg = d[:, 0]
    return lax.rsqrt(jnp.maximum(deg, 1.0))


def _tc1_body(dop_ref, x_ref, w_ref, h_ref):
    s_out = _norm_from_partials(dop_ref)
    h_ref[...] = jnp.dot(
        x_ref[...] * s_out[:, None], w_ref[...],
        preferred_element_type=jnp.float32,
    )


def _tc1(dop, x, w):
    return pl.pallas_call(
        _tc1_body,
        grid=(GRID,),
        in_specs=[
            pl.BlockSpec((NC, BLK, 16), lambda i: (0, i, 0)),
            pl.BlockSpec((BLK, D), lambda i: (i, 0)),
            pl.BlockSpec((D, D), lambda i: (0, 0)),
        ],
        out_specs=pl.BlockSpec((BLK, D), lambda i: (i, 0)),
        out_shape=jax.ShapeDtypeStruct((N, D), jnp.float32),
    )(dop, x, w)


def _tc2_body(p_ref, dip_ref, dop_ref, b1_ref, w_ref, o_ref):
    s_in = _norm_from_partials(dip_ref)
    s_out = _norm_from_partials(dop_ref)
    agg = p_ref[0] + p_ref[1]
    h = jnp.maximum(agg * s_in[:, None] + b1_ref[...], 0.0)
    o_ref[...] = jnp.dot(
        h * s_out[:, None], w_ref[...], preferred_element_type=jnp.float32
    )


def _tc2(p, dip, dop, b1, w):
    return pl.pallas_call(
        _tc2_body,
        grid=(GRID,),
        in_specs=[
            pl.BlockSpec((NC, BLK, D), lambda i: (0, i, 0)),
            pl.BlockSpec((NC, BLK, 16), lambda i: (0, i, 0)),
            pl.BlockSpec((NC, BLK, 16), lambda i: (0, i, 0)),
            pl.BlockSpec((1, D), lambda i: (0, 0)),
            pl.BlockSpec((D, D), lambda i: (0, 0)),
        ],
        out_specs=pl.BlockSpec((BLK, D), lambda i: (i, 0)),
        out_shape=jax.ShapeDtypeStruct((N, D), jnp.float32),
    )(p, dip, dop, b1, w)


def _tc3_body(q_ref, dip_ref, b2_ref, o_ref):
    s_in = _norm_from_partials(dip_ref)
    agg = q_ref[0] + q_ref[1]
    o_ref[...] = agg * s_in[:, None] + b2_ref[...]


def _tc3(q, dip, b2):
    return pl.pallas_call(
        _tc3_body,
        grid=(GRID,),
        in_specs=[
            pl.BlockSpec((NC, BLK, D), lambda i: (0, i, 0)),
            pl.BlockSpec((NC, BLK, 16), lambda i: (0, i, 0)),
            pl.BlockSpec((1, D), lambda i: (0, 0)),
        ],
        out_specs=pl.BlockSpec((BLK, D), lambda i: (i, 0)),
        out_shape=jax.ShapeDtypeStruct((N, D), jnp.float32),
    )(q, dip, b2)


def kernel(x, edge_index, W1, b1, W2, b2):
    src = edge_index[0]
    dst = edge_index[1]
    dop, dip = _sc_degrees(src, dst)
    h1 = _tc1(dop, x, W1)
    p1 = _sc_agg(h1, src, dst)
    t2 = _tc2(p1, dip, dop, b1.reshape(1, D), W2)
    p2 = _sc_agg(t2, src, dst)
    out = _tc3(p2, dip, b2.reshape(1, D))
    return out


# speedup vs baseline: 4.4437x; 4.4437x over previous
import functools

import jax
import jax.numpy as jnp
from jax import lax
from jax.experimental import pallas as pl
from jax.experimental.pallas import tpu as pltpu
from jax.experimental.pallas import tpu_sc as plsc

N = 10000
E = 320000
D = 128
NP = 10240

NC = 2
NS = 16
NW = NC * NS

EPT = E // NW
CH = 80
NCHUNK = EPT // CH
RPT = NP // NS

_mesh = plsc.VectorSubcoreMesh(core_axis_name="c", subcore_axis_name="s")


def _fill_idx(zidx, base, iota16):
    @pl.loop(0, CH, step=16)
    def _(j):
        zidx[pl.ds(j, 16)] = base + j + iota16


def _sc_degrees(src, dst):
    @functools.partial(
        pl.kernel,
        out_type=[
            jax.ShapeDtypeStruct((NC, NP, 16), jnp.float32),
            jax.ShapeDtypeStruct((NC, NP, 16), jnp.float32),
        ],
        mesh=_mesh,
        scratch_types=[
            pltpu.VMEM((CH,), jnp.int32),
            pltpu.VMEM((CH,), jnp.int32),
            pltpu.VMEM((CH,), jnp.int32),
            pltpu.VMEM((CH, 16), jnp.float32),
            pltpu.VMEM((CH, 16), jnp.float32),
            pltpu.VMEM((CH, 16), jnp.float32),
            pltpu.VMEM_SHARED((NP, 16), jnp.float32),
            pltpu.VMEM_SHARED((NP, 16), jnp.float32),
        ],
    )
    def k(src_h, dst_h, dop_h, dip_h,
          sidx, didx, zidx, ones_v, zrows, grows, acc_o, acc_i):
        c = lax.axis_index("c")
        s = lax.axis_index("s")
        wid = c * NS + s
        ebase = wid * EPT
        r0 = s * RPT
        iota16 = lax.iota(jnp.int32, 16)

        zero16 = jnp.zeros((16,), jnp.float32)
        one16 = jnp.full((16,), 1.0, jnp.float32)

        @pl.loop(0, CH)
        def _(i):
            ones_v[i, :] = one16
            zrows[i, :] = zero16

        @pl.loop(0, RPT, step=CH)
        def _(k0):
            _fill_idx(zidx, r0 + k0, iota16)
            pltpu.sync_copy(zrows, acc_o.at[zidx])
            pltpu.sync_copy(zrows, acc_i.at[zidx])

        plsc.subcore_barrier()

        @pl.loop(0, NCHUNK)
        def _(ci):
            base = ebase + ci * CH
            pltpu.sync_copy(src_h.at[pl.ds(base, CH)], sidx)
            pltpu.sync_copy(dst_h.at[pl.ds(base, CH)], didx)
            pltpu.sync_copy(ones_v, acc_o.at[sidx], add=True)
            pltpu.sync_copy(ones_v, acc_i.at[didx], add=True)

        plsc.subcore_barrier()

        @pl.loop(0, RPT, step=CH)
        def _(k0):
            _fill_idx(zidx, r0 + k0, iota16)
            pltpu.sync_copy(acc_o.at[zidx], grows)
            pltpu.sync_copy(grows, dop_h.at[c, pl.ds(r0 + k0, CH), :])
            pltpu.sync_copy(acc_i.at[zidx], grows)
            pltpu.sync_copy(grows, dip_h.at[c, pl.ds(r0 + k0, CH), :])

    return k(src, dst)


def _sc_agg(h, src, dst):
    @functools.partial(
        pl.kernel,
        out_type=jax.ShapeDtypeStruct((NC, NP, D), jnp.float32),
        mesh=_mesh,
        scratch_types=[
            pltpu.VMEM((CH,), jnp.int32),
            pltpu.VMEM((CH,), jnp.int32),
            pltpu.VMEM((CH,), jnp.int32),
            pltpu.VMEM((CH, D), jnp.float32),
            pltpu.VMEM((CH, D), jnp.float32),
            pltpu.VMEM_SHARED((NP, D), jnp.float32),
            pltpu.SemaphoreType.DMA,
        ],
    )
    def k(h_h, src_h, dst_h, out_h, sidx, didx, zidx, rows, zrows, acc, sem):
        c = lax.axis_index("c")
        s = lax.axis_index("s")
        wid = c * NS + s
        ebase = wid * EPT
        r0 = s * RPT
        iota16 = lax.iota(jnp.int32, 16)

        zero16 = jnp.zeros((16,), jnp.float32)

        @pl.loop(0, CH)
        def _(i):
            @pl.loop(0, D, step=16)
            def _(j):
                zrows[i, pl.ds(j, 16)] = zero16

        @pl.loop(0, RPT, step=CH)
        def _(k0):
            _fill_idx(zidx, r0 + k0, iota16)
            pltpu.sync_copy(zrows, acc.at[zidx])

        plsc.subcore_barrier()

        @pl.loop(0, NCHUNK)
        def _(ci):
            base = ebase + ci * CH
            pltpu.sync_copy(src_h.at[pl.ds(base, CH)], sidx)
            pltpu.sync_copy(dst_h.at[pl.ds(base, CH)], didx)
            pltpu.async_copy(h_h.at[sidx], rows, sem).wait()
            pltpu.sync_copy(rows, acc.at[didx], add=True)

        plsc.subcore_barrier()

        @pl.loop(0, RPT, step=CH)
        def _(k0):
            _fill_idx(zidx, r0 + k0, iota16)
            pltpu.sync_copy(acc.at[zidx], rows)
            pltpu.sync_copy(rows, out_h.at[c, pl.ds(r0 + k0, CH), :])

    return k(h, src, dst)


BLK = 1024
GRID = NP // BLK


def _norm_from_partials(p_ref):
    d = p_ref[0] + p_ref[1]
    deg = d[:, 0]
    return lax.rsqrt(jnp.maximum(deg, 1.0))


def _tc1_body(dop_ref, x_ref, w_ref, h_ref):
    s_out = _norm_from_partials(dop_ref)
    h_ref[...] = jnp.dot(
        x_ref[...] * s_out[:, None], w_ref[...],
        preferred_element_type=jnp.float32,
    )


def _tc1(dop, x, w):
    return pl.pallas_call(
        _tc1_body,
        grid=(GRID,),
        in_specs=[
            pl.BlockSpec((NC, BLK, 16), lambda i: (0, i, 0)),
            pl.BlockSpec((BLK, D), lambda i: (i, 0)),
            pl.BlockSpec((D, D), lambda i: (0, 0)),
        ],
        out_specs=pl.BlockSpec((BLK, D), lambda i: (i, 0)),
        out_shape=jax.ShapeDtypeStruct((NP, D), jnp.float32),
    )(dop, x, w)


def _tc2_body(p_ref, dip_ref, dop_ref, b1_ref, w_ref, o_ref):
    s_in = _norm_from_partials(dip_ref)
    s_out = _norm_from_partials(dop_ref)
    agg = p_ref[0] + p_ref[1]
    h = jnp.maximum(agg * s_in[:, None] + b1_ref[...], 0.0)
    o_ref[...] = jnp.dot(
        h * s_out[:, None], w_ref[...], preferred_element_type=jnp.float32
    )


def _tc2(p, dip, dop, b1, w):
    return pl.pallas_call(
        _tc2_body,
        grid=(GRID,),
        in_specs=[
            pl.BlockSpec((NC, BLK, D), lambda i: (0, i, 0)),
            pl.BlockSpec((NC, BLK, 16), lambda i: (0, i, 0)),
            pl.BlockSpec((NC, BLK, 16), lambda i: (0, i, 0)),
            pl.BlockSpec((1, D), lambda i: (0, 0)),
            pl.BlockSpec((D, D), lambda i: (0, 0)),
        ],
        out_specs=pl.BlockSpec((BLK, D), lambda i: (i, 0)),
        out_shape=jax.ShapeDtypeStruct((NP, D), jnp.float32),
    )(p, dip, dop, b1, w)


def _tc3_body(q_ref, dip_ref, b2_ref, o_ref):
    s_in = _norm_from_partials(dip_ref)
    agg = q_ref[0] + q_ref[1]
    o_ref[...] = agg * s_in[:, None] + b2_ref[...]


def _tc3(q, dip, b2):
    return pl.pallas_call(
        _tc3_body,
        grid=(GRID,),
        in_specs=[
            pl.BlockSpec((NC, BLK, D), lambda i: (0, i, 0)),
            pl.BlockSpec((NC, BLK, 16), lambda i: (0, i, 0)),
            pl.BlockSpec((1, D), lambda i: (0, 0)),
        ],
        out_specs=pl.BlockSpec((BLK, D), lambda i: (i, 0)),
        out_shape=jax.ShapeDtypeStruct((N, D), jnp.float32),
    )(q, dip, b2)


def kernel(x, edge_index, W1, b1, W2, b2):
    src = edge_index[0]
    dst = edge_index[1]
    dop, dip = _sc_degrees(src, dst)
    h1 = _tc1(dop, x, W1)
    p1 = _sc_agg(h1, src, dst)
    t2 = _tc2(p1, dip, dop, b1.reshape(1, D), W2)
    p2 = _sc_agg(t2, src, dst)
    out = _tc3(p2, dip, b2.reshape(1, D))
    return out
